# baseline (speedup 1.0000x reference)
import jax
import jax.numpy as jnp
from jax import lax
from jax.experimental import pallas as pl
from jax.experimental.pallas import tpu as pltpu

_IOU = 0.5
_BIG = 1e9
_HMAX = 128


def _merge_kernel(j_ref, sr_ref, c_ref, d_ref, p_ref, mcd_ref, pn_ref, m_ref,
                  cnt_ref, mc_ref, cntc_ref):
    N = j_ref.shape[0]
    BLK = min(256, N)
    sr = sr_ref[...]
    icol = lax.broadcasted_iota(jnp.int32, (_HMAX, 1), 0)
    ibig = jnp.int32(1 << 30)

    def cond(c):
        return jnp.max(c[0]) > 0.0

    def body(c):
        u, f, hix, k = c
        ub = u > 0.0
        key = jnp.where(ub, sr, -1.0)
        idx = jnp.argmax(key).astype(jnp.int32)
        jrow = j_ref[pl.ds(idx, 1), :]
        newc = ub & (jrow >= _IOU)
        f = jnp.where(newc, idx, f)
        u = jnp.where(newc, 0.0, u)
        hix = jnp.where(icol == k, idx, hix)
        return u, f, hix, k + 1

    _, f, hix, nheads = lax.while_loop(
        cond, body,
        (jnp.ones((1, N), jnp.float32), jnp.full((1, N), -1, jnp.int32),
         jnp.full((_HMAX, 1), ibig, jnp.int32), jnp.int32(0)))

    @pl.when(nheads <= _HMAX)
    def _compact():
        mcf = jnp.where(hix == f, 1.0, 0.0)
        cntc_ref[...] = jnp.sum(mcf, axis=1, keepdims=True)
        mc_ref[...] = mcf.astype(jnp.bfloat16)
        cnt = cntc_ref[...]
        ssc = jnp.dot(mc_ref[...], c_ref[...].astype(jnp.bfloat16),
                      preferred_element_type=jnp.float32)
        ssd = jnp.dot(mc_ref[...], d_ref[...].astype(jnp.bfloat16),
                      preferred_element_type=jnp.float32)
        inv = jnp.where(cnt > 0.0, 1.0 / jnp.maximum(cnt, 1.0), 0.0)
        avg = jnp.concatenate(
            [(ssc * inv).astype(jnp.bfloat16),
             (ssd * inv).astype(jnp.bfloat16)], axis=1)
        mcd_ref[...] = lax.dot_general(
            mc_ref[...], avg, (((0,), (0,)), ((), ())),
            preferred_element_type=jnp.float32)
        max_clique = jnp.max(cnt).astype(jnp.int32)
        min_clique = jnp.min(
            jnp.where(cnt > 0.0, cnt, _BIG)).astype(jnp.int32)
        pn_ref[6] = p_ref[6] + max_clique
        pn_ref[7] = p_ref[7] + min_clique

    @pl.when(nheads > _HMAX)
    def _full():
        def build_blk(b, _):
            i0 = pl.multiple_of(b * BLK, BLK)
            icb = lax.broadcasted_iota(jnp.int32, (BLK, 1), 0) + i0
            mf = jnp.where(icb == f, 1.0, 0.0)
            cnt_ref[pl.ds(i0, BLK), :] = jnp.sum(mf, axis=1, keepdims=True)
            m_ref[pl.ds(i0, BLK), :] = mf.astype(jnp.bfloat16)
            return 0
        lax.fori_loop(0, N // BLK, build_blk, 0)

        cnt = cnt_ref[...]
        ssc = jnp.dot(m_ref[...], c_ref[...].astype(jnp.bfloat16),
                      preferred_element_type=jnp.float32)
        ssd = jnp.dot(m_ref[...], d_ref[...].astype(jnp.bfloat16),
                      preferred_element_type=jnp.float32)
        inv = jnp.where(cnt > 0.0, 1.0 / jnp.maximum(cnt, 1.0), 0.0)
        avg = jnp.concatenate(
            [(ssc * inv).astype(jnp.bfloat16),
             (ssd * inv).astype(jnp.bfloat16)], axis=1)
        mcd_ref[...] = lax.dot_general(
            m_ref[...], avg, (((0,), (0,)), ((), ())),
            preferred_element_type=jnp.float32)
        max_clique = jnp.max(cnt).astype(jnp.int32)
        min_clique = jnp.min(
            jnp.where(cnt > 0.0, cnt, _BIG)).astype(jnp.int32)
        pn_ref[6] = p_ref[6] + max_clique
        pn_ref[7] = p_ref[7] + min_clique

    pn_ref[0] = p_ref[0]
    pn_ref[1] = p_ref[1]
    pn_ref[2] = p_ref[2] + 1
    pn_ref[3] = p_ref[3]
    pn_ref[4] = p_ref[4]
    pn_ref[5] = p_ref[5] + nheads


def _merge_pallas(J, sr, C, D, P):
    N, K = C.shape
    vmem_limit = int(min(
        2 * N * N * 4 + N * N * 2 + 16 * N * K * 4 + (4 << 20), 60 << 20))
    out_shape = (
        jax.ShapeDtypeStruct((N, 2 * K), jnp.float32),
        jax.ShapeDtypeStruct((8,), jnp.int32),
    )
    return pl.pallas_call(
        _merge_kernel,
        out_shape=out_shape,
        grid=(1,),
        in_specs=[
            pl.BlockSpec((N, N), lambda i: (0, 0)),
            pl.BlockSpec((1, N), lambda i: (0, 0)),
            pl.BlockSpec((N, K), lambda i: (0, 0)),
            pl.BlockSpec((N, K), lambda i: (0, 0)),
            pl.BlockSpec(memory_space=pltpu.SMEM),
        ],
        out_specs=(
            pl.BlockSpec((N, 2 * K), lambda i: (0, 0)),
            pl.BlockSpec(memory_space=pltpu.SMEM),
        ),
        scratch_shapes=[
            pltpu.VMEM((N, N), jnp.bfloat16),
            pltpu.VMEM((N, 1), jnp.float32),
            pltpu.VMEM((_HMAX, N), jnp.bfloat16),
            pltpu.VMEM((_HMAX, 1), jnp.float32),
        ],
        compiler_params=pltpu.CompilerParams(
            dimension_semantics=("arbitrary",),
            vmem_limit_bytes=vmem_limit),
    )(J, sr, C, D, P)


def kernel(S, J, C, D, P):
    N = S.shape[0]
    K = C.shape[1]

    sr = S.astype(jnp.float32).reshape(1, N)
    MCD, P_new = _merge_pallas(
        J.astype(jnp.float32), sr, C.astype(jnp.float32),
        D.astype(jnp.float32), P)

    MC = MCD[:, :K].astype(C.dtype)
    MD = MCD[:, K:].astype(D.dtype)
    return MC, MD, P_new


# speedup vs baseline: 1.1382x; 1.1382x over previous
import jax
import jax.numpy as jnp
from jax import lax
from jax.experimental import pallas as pl
from jax.experimental.pallas import tpu as pltpu

_IOU = 0.5
_BIG = 1e9
_HMAX = 128


def _merge_kernel(j_ref, sr_ref, cd_ref, p_ref, mcd_ref, pn_ref, m_ref,
                  cnt_ref, mc_ref, cntc_ref):
    N = j_ref.shape[0]
    BLK = min(256, N)
    sr = sr_ref[...]
    icol = lax.broadcasted_iota(jnp.int32, (_HMAX, 1), 0)
    ibig = jnp.int32(1 << 30)

    def cond(c):
        return jnp.max(c[0]) >= 0.0

    def body(c):
        ks, f, hix, k = c
        idx = jnp.argmax(ks).astype(jnp.int32)
        jrow = j_ref[pl.ds(idx, 1), :]
        newc = (ks >= 0.0) & (jrow >= _IOU)
        f = jnp.where(newc, idx, f)
        ks = jnp.where(newc, -1.0, ks)
        hix = jnp.where(icol == k, idx, hix)
        return ks, f, hix, k + 1

    _, f, hix, nheads = lax.while_loop(
        cond, body,
        (sr, jnp.full((1, N), -1, jnp.int32),
         jnp.full((_HMAX, 1), ibig, jnp.int32), jnp.int32(0)))

    @pl.when(nheads <= _HMAX)
    def _compact():
        mcf = jnp.where(hix == f, 1.0, 0.0)
        cntc_ref[...] = jnp.sum(mcf, axis=1, keepdims=True)
        mc_ref[...] = mcf.astype(jnp.bfloat16)
        cnt = cntc_ref[...]
        ssum = jnp.dot(mc_ref[...], cd_ref[...],
                       preferred_element_type=jnp.float32)
        inv = jnp.where(cnt > 0.0, 1.0 / jnp.maximum(cnt, 1.0), 0.0)
        avg = (ssum * inv).astype(jnp.bfloat16)
        mcd_ref[...] = lax.dot_general(
            mc_ref[...], avg, (((0,), (0,)), ((), ())),
            preferred_element_type=jnp.float32)
        max_clique = jnp.max(cnt).astype(jnp.int32)
        min_clique = jnp.min(
            jnp.where(cnt > 0.0, cnt, _BIG)).astype(jnp.int32)
        pn_ref[6] = p_ref[6] + max_clique
        pn_ref[7] = p_ref[7] + min_clique

    @pl.when(nheads > _HMAX)
    def _full():
        def build_blk(b, _):
            i0 = pl.multiple_of(b * BLK, BLK)
            icb = lax.broadcasted_iota(jnp.int32, (BLK, 1), 0) + i0
            mf = jnp.where(icb == f, 1.0, 0.0)
            cnt_ref[pl.ds(i0, BLK), :] = jnp.sum(mf, axis=1, keepdims=True)
            m_ref[pl.ds(i0, BLK), :] = mf.astype(jnp.bfloat16)
            return 0
        lax.fori_loop(0, N // BLK, build_blk, 0)

        cnt = cnt_ref[...]
        ssum = jnp.dot(m_ref[...], cd_ref[...],
                       preferred_element_type=jnp.float32)
        inv = jnp.where(cnt > 0.0, 1.0 / jnp.maximum(cnt, 1.0), 0.0)
        avg = (ssum * inv).astype(jnp.bfloat16)
        mcd_ref[...] = lax.dot_general(
            m_ref[...], avg, (((0,), (0,)), ((), ())),
            preferred_element_type=jnp.float32)
        max_clique = jnp.max(cnt).astype(jnp.int32)
        min_clique = jnp.min(
            jnp.where(cnt > 0.0, cnt, _BIG)).astype(jnp.int32)
        pn_ref[6] = p_ref[6] + max_clique
        pn_ref[7] = p_ref[7] + min_clique

    pn_ref[0] = p_ref[0]
    pn_ref[1] = p_ref[1]
    pn_ref[2] = p_ref[2] + 1
    pn_ref[3] = p_ref[3]
    pn_ref[4] = p_ref[4]
    pn_ref[5] = p_ref[5] + nheads


def _merge_pallas(J, sr, CD, P):
    N, K2 = CD.shape
    vmem_limit = int(min(
        2 * N * N * 4 + N * N * 2 + 8 * N * K2 * 4 + (4 << 20), 60 << 20))
    out_shape = (
        jax.ShapeDtypeStruct((N, K2), jnp.float32),
        jax.ShapeDtypeStruct((8,), jnp.int32),
    )
    return pl.pallas_call(
        _merge_kernel,
        out_shape=out_shape,
        grid=(1,),
        in_specs=[
            pl.BlockSpec((N, N), lambda i: (0, 0)),
            pl.BlockSpec((1, N), lambda i: (0, 0)),
            pl.BlockSpec((N, K2), lambda i: (0, 0)),
            pl.BlockSpec(memory_space=pltpu.SMEM),
        ],
        out_specs=(
            pl.BlockSpec((N, K2), lambda i: (0, 0)),
            pl.BlockSpec(memory_space=pltpu.SMEM),
        ),
        scratch_shapes=[
            pltpu.VMEM((N, N), jnp.bfloat16),
            pltpu.VMEM((N, 1), jnp.float32),
            pltpu.VMEM((_HMAX, N), jnp.bfloat16),
            pltpu.VMEM((_HMAX, 1), jnp.float32),
        ],
        compiler_params=pltpu.CompilerParams(
            dimension_semantics=("arbitrary",),
            vmem_limit_bytes=vmem_limit),
    )(J, sr, CD, P)


def kernel(S, J, C, D, P):
    N = S.shape[0]
    K = C.shape[1]

    sr = S.astype(jnp.float32).reshape(1, N)
    CD = jnp.concatenate(
        [C.astype(jnp.bfloat16), D.astype(jnp.bfloat16)], axis=1)

    MCD, P_new = _merge_pallas(J.astype(jnp.float32), sr, CD, P)

    MC = MCD[:, :K].astype(C.dtype)
    MD = MCD[:, K:].astype(D.dtype)
    return MC, MD, P_new
